# baseline (speedup 1.0000x reference)
import jax
import jax.numpy as jnp
import numpy as np
from jax.experimental import pallas as pl
from jax.experimental.pallas import tpu as pltpu

_BLOCK = 32
_NLOCAL = 2
_STRIDE = 8
_TQ = 128
_SUPER = _STRIDE * _BLOCK
_LOCW = _TQ + 2 * _BLOCK
_NEG = -1e30


def _local_start(t, S):
    return min(max(t * _TQ - _BLOCK, 0), S - _LOCW)


def _make_biases(T, S):
    ns = (S // _SUPER) * _BLOCK
    rows = np.arange(T)[:, None] // _BLOCK
    cs = np.arange(ns)[None, :] // _BLOCK * _STRIDE
    bias_s = np.where(np.abs(rows - cs) >= _NLOCAL, 0.0, _NEG).astype(np.float32)

    bias_l = np.full((T, _LOCW), _NEG, dtype=np.float32)
    for t in range(T // _TQ):
        start = _local_start(t, S)
        r = np.arange(t * _TQ, (t + 1) * _TQ)[:, None] // _BLOCK
        c = start // _BLOCK + np.arange(_LOCW)[None, :] // _BLOCK
        bias_l[t * _TQ:(t + 1) * _TQ] = np.where(
            np.abs(r - c) < _NLOCAL, 0.0, _NEG)
    return bias_s, bias_l


def _attn_kernel(q_ref, k_ref, v_ref, bs_ref, bl_ref, o_ref, ks_ref, vs_ref):
    S = k_ref.shape[1]
    n_super = S // _SUPER

    for i in range(n_super):
        ks_ref[i * _BLOCK:(i + 1) * _BLOCK, :] = \
            k_ref[0, i * _SUPER:i * _SUPER + _BLOCK, :]
        vs_ref[i * _BLOCK:(i + 1) * _BLOCK, :] = \
            v_ref[0, i * _SUPER:i * _SUPER + _BLOCK, :]
    ks = ks_ref[...]
    vs = vs_ref[...]

    dn = (((1,), (1,)), ((), ()))
    dv = (((1,), (0,)), ((), ()))
    for t in range(q_ref.shape[1] // _TQ):
        q = q_ref[0, t * _TQ:(t + 1) * _TQ, :]
        start = _local_start(t, S)
        kl = k_ref[0, start:start + _LOCW, :]
        vl = v_ref[0, start:start + _LOCW, :]

        ss = jax.lax.dot_general(q, ks, dn, preferred_element_type=jnp.float32)
        ss = ss + bs_ref[t * _TQ:(t + 1) * _TQ, :]
        sl = jax.lax.dot_general(q, kl, dn, preferred_element_type=jnp.float32)
        sl = sl + bl_ref[t * _TQ:(t + 1) * _TQ, :]

        m = jnp.maximum(jnp.max(ss, axis=1), jnp.max(sl, axis=1))
        ps = jnp.exp(ss - m[:, None])
        plc = jnp.exp(sl - m[:, None])
        denom = jnp.sum(ps, axis=1) + jnp.sum(plc, axis=1)

        out = jax.lax.dot_general(ps, vs, dv, preferred_element_type=jnp.float32)
        out = out + jax.lax.dot_general(plc, vl, dv,
                                        preferred_element_type=jnp.float32)
        o_ref[0, t * _TQ:(t + 1) * _TQ, :] = out / denom[:, None]


def kernel(query, key, value):
    B, T, H, E = query.shape
    S = key.shape[1]
    temp = 1.0 / float(np.sqrt(E))
    q = jnp.transpose(query[0], (1, 0, 2)) * temp
    k = jnp.transpose(key[0], (1, 0, 2))
    v = jnp.transpose(value[0], (1, 0, 2))
    ns = (S // _SUPER) * _BLOCK
    bias_s, bias_l = _make_biases(T, S)

    out = pl.pallas_call(
        _attn_kernel,
        grid=(H,),
        in_specs=[
            pl.BlockSpec((1, T, E), lambda h: (h, 0, 0)),
            pl.BlockSpec((1, S, E), lambda h: (h, 0, 0)),
            pl.BlockSpec((1, S, E), lambda h: (h, 0, 0)),
            pl.BlockSpec((T, ns), lambda h: (0, 0)),
            pl.BlockSpec((T, _LOCW), lambda h: (0, 0)),
        ],
        out_specs=pl.BlockSpec((1, _TQ * (T // _TQ), E), lambda h: (h, 0, 0)),
        out_shape=jax.ShapeDtypeStruct((H, T, E), jnp.float32),
        scratch_shapes=[
            pltpu.VMEM((ns, E), jnp.float32),
            pltpu.VMEM((ns, E), jnp.float32),
        ],
        compiler_params=pltpu.CompilerParams(
            dimension_semantics=("parallel",),
        ),
    )(q, k, v, jnp.asarray(bias_s), jnp.asarray(bias_l))
    return jnp.transpose(out, (1, 0, 2))[None]


# speedup vs baseline: 1.2394x; 1.2394x over previous
import jax
import jax.numpy as jnp
import numpy as np
from jax.experimental import pallas as pl
from jax.experimental.pallas import tpu as pltpu

_BLOCK = 32
_NLOCAL = 2
_STRIDE = 8
_TQ = 256
_SUPER = _STRIDE * _BLOCK
_LOCW = _TQ + 2 * _BLOCK
_NEG = -1e30


def _local_start(t, S):
    return min(max(t * _TQ - _BLOCK, 0), S - _LOCW)


def _make_biases(T, S):
    ns = (S // _SUPER) * _BLOCK
    rows = np.arange(T)[:, None] // _BLOCK
    cs = np.arange(ns)[None, :] // _BLOCK * _STRIDE
    bias_s = np.where(np.abs(rows - cs) >= _NLOCAL, 0.0, _NEG).astype(np.float32)

    bias_l = np.full((T, _LOCW), _NEG, dtype=np.float32)
    for t in range(T // _TQ):
        start = _local_start(t, S)
        r = np.arange(t * _TQ, (t + 1) * _TQ)[:, None] // _BLOCK
        c = start // _BLOCK + np.arange(_LOCW)[None, :] // _BLOCK
        bias_l[t * _TQ:(t + 1) * _TQ] = np.where(
            np.abs(r - c) < _NLOCAL, 0.0, _NEG)
    return bias_s, bias_l


def _attn_kernel(q_ref, k_ref, v_ref, bs_ref, bl_ref, o_ref, ks_ref, vs_ref):
    S = k_ref.shape[1]
    n_super = S // _SUPER

    for i in range(n_super):
        ks_ref[i * _BLOCK:(i + 1) * _BLOCK, :] = \
            k_ref[0, i * _SUPER:i * _SUPER + _BLOCK, :]
        vs_ref[i * _BLOCK:(i + 1) * _BLOCK, :] = \
            v_ref[0, i * _SUPER:i * _SUPER + _BLOCK, :]
    ks = ks_ref[...]
    vs = vs_ref[...]

    dn = (((1,), (1,)), ((), ()))
    dv = (((1,), (0,)), ((), ()))
    for t in range(q_ref.shape[1] // _TQ):
        q = q_ref[0, t * _TQ:(t + 1) * _TQ, :]
        start = _local_start(t, S)
        kl = k_ref[0, start:start + _LOCW, :]
        vl = v_ref[0, start:start + _LOCW, :]

        ss = jax.lax.dot_general(q, ks, dn, preferred_element_type=jnp.float32)
        ss = ss + bs_ref[t * _TQ:(t + 1) * _TQ, :]
        sl = jax.lax.dot_general(q, kl, dn, preferred_element_type=jnp.float32)
        sl = sl + bl_ref[t * _TQ:(t + 1) * _TQ, :]

        m = jnp.maximum(jnp.max(ss, axis=1), jnp.max(sl, axis=1))
        ps = jnp.exp(ss - m[:, None])
        plc = jnp.exp(sl - m[:, None])
        denom = jnp.sum(ps, axis=1) + jnp.sum(plc, axis=1)

        out = jax.lax.dot_general(ps, vs, dv, preferred_element_type=jnp.float32)
        out = out + jax.lax.dot_general(plc, vl, dv,
                                        preferred_element_type=jnp.float32)
        o_ref[0, t * _TQ:(t + 1) * _TQ, :] = out / denom[:, None]


def kernel(query, key, value):
    B, T, H, E = query.shape
    S = key.shape[1]
    temp = 1.0 / float(np.sqrt(E))
    q = jnp.transpose(query[0], (1, 0, 2)) * temp
    k = jnp.transpose(key[0], (1, 0, 2))
    v = jnp.transpose(value[0], (1, 0, 2))
    ns = (S // _SUPER) * _BLOCK
    bias_s, bias_l = _make_biases(T, S)

    out = pl.pallas_call(
        _attn_kernel,
        grid=(H,),
        in_specs=[
            pl.BlockSpec((1, T, E), lambda h: (h, 0, 0)),
            pl.BlockSpec((1, S, E), lambda h: (h, 0, 0)),
            pl.BlockSpec((1, S, E), lambda h: (h, 0, 0)),
            pl.BlockSpec((T, ns), lambda h: (0, 0)),
            pl.BlockSpec((T, _LOCW), lambda h: (0, 0)),
        ],
        out_specs=pl.BlockSpec((1, _TQ * (T // _TQ), E), lambda h: (h, 0, 0)),
        out_shape=jax.ShapeDtypeStruct((H, T, E), jnp.float32),
        scratch_shapes=[
            pltpu.VMEM((ns, E), jnp.float32),
            pltpu.VMEM((ns, E), jnp.float32),
        ],
        compiler_params=pltpu.CompilerParams(
            dimension_semantics=("parallel",),
        ),
    )(q, k, v, jnp.asarray(bias_s), jnp.asarray(bias_l))
    return jnp.transpose(out, (1, 0, 2))[None]


# speedup vs baseline: 1.5253x; 1.2307x over previous
import jax
import jax.numpy as jnp
import numpy as np
from jax.experimental import pallas as pl
from jax.experimental.pallas import tpu as pltpu

_BLOCK = 32
_NLOCAL = 2
_STRIDE = 8
_TQ = 256
_SUPER = _STRIDE * _BLOCK
_LOCW = _TQ + 2 * _BLOCK
_NEG = -1e30


def _local_start(t, S):
    return min(max(t * _TQ - _BLOCK, 0), S - _LOCW)


def _make_biases(T, S):
    ns = (S // _SUPER) * _BLOCK
    rows = np.arange(T)[:, None] // _BLOCK
    cs = np.arange(ns)[None, :] // _BLOCK * _STRIDE
    bias_s = np.where(np.abs(rows - cs) >= _NLOCAL, 0.0, _NEG).astype(np.float32)

    bias_l = np.full((T, _LOCW), _NEG, dtype=np.float32)
    for t in range(T // _TQ):
        start = _local_start(t, S)
        r = np.arange(t * _TQ, (t + 1) * _TQ)[:, None] // _BLOCK
        c = start // _BLOCK + np.arange(_LOCW)[None, :] // _BLOCK
        bias_l[t * _TQ:(t + 1) * _TQ] = np.where(
            np.abs(r - c) < _NLOCAL, 0.0, _NEG)
    return bias_s, bias_l


def _attn_kernel(H, E, q_ref, k_ref, v_ref, bs_ref, bl_ref, o_ref,
                 ks_ref, vs_ref):
    t = pl.program_id(0)
    S = k_ref.shape[0]
    n_super = S // _SUPER
    temp = 1.0 / float(np.sqrt(E))

    @pl.when(t == 0)
    def _gather():
        for i in range(n_super):
            ks_ref[i * _BLOCK:(i + 1) * _BLOCK, :] = \
                k_ref[i * _SUPER:i * _SUPER + _BLOCK, :]
            vs_ref[i * _BLOCK:(i + 1) * _BLOCK, :] = \
                v_ref[i * _SUPER:i * _SUPER + _BLOCK, :]

    start = pl.multiple_of(jnp.clip(t * _TQ - _BLOCK, 0, S - _LOCW), _BLOCK)
    bs = bs_ref[...]
    bl = bl_ref[...]

    dn = (((1,), (1,)), ((), ()))
    dv = (((1,), (0,)), ((), ()))
    for h in range(H):
        sl_h = slice(h * E, (h + 1) * E)
        q = q_ref[:, sl_h] * temp
        ks = ks_ref[:, sl_h]
        vs = vs_ref[:, sl_h]
        kl = k_ref[pl.ds(start, _LOCW), sl_h]
        vl = v_ref[pl.ds(start, _LOCW), sl_h]

        ss = jax.lax.dot_general(q, ks, dn,
                                 preferred_element_type=jnp.float32) + bs
        sl = jax.lax.dot_general(q, kl, dn,
                                 preferred_element_type=jnp.float32) + bl

        m = jnp.maximum(jnp.max(ss, axis=1), jnp.max(sl, axis=1))
        ps = jnp.exp(ss - m[:, None])
        plc = jnp.exp(sl - m[:, None])
        denom = jnp.sum(ps, axis=1) + jnp.sum(plc, axis=1)

        out = jax.lax.dot_general(ps, vs, dv,
                                  preferred_element_type=jnp.float32)
        out = out + jax.lax.dot_general(plc, vl, dv,
                                        preferred_element_type=jnp.float32)
        o_ref[:, sl_h] = out / denom[:, None]


def kernel(query, key, value):
    B, T, H, E = query.shape
    S = key.shape[1]
    q = query[0].reshape(T, H * E)
    k = key[0].reshape(S, H * E)
    v = value[0].reshape(S, H * E)
    ns = (S // _SUPER) * _BLOCK
    bias_s, bias_l = _make_biases(T, S)

    import functools
    out = pl.pallas_call(
        functools.partial(_attn_kernel, H, E),
        grid=(T // _TQ,),
        in_specs=[
            pl.BlockSpec((_TQ, H * E), lambda t: (t, 0)),
            pl.BlockSpec((S, H * E), lambda t: (0, 0)),
            pl.BlockSpec((S, H * E), lambda t: (0, 0)),
            pl.BlockSpec((_TQ, ns), lambda t: (t, 0)),
            pl.BlockSpec((_TQ, _LOCW), lambda t: (t, 0)),
        ],
        out_specs=pl.BlockSpec((_TQ, H * E), lambda t: (t, 0)),
        out_shape=jax.ShapeDtypeStruct((T, H * E), jnp.float32),
        scratch_shapes=[
            pltpu.VMEM((ns, H * E), jnp.float32),
            pltpu.VMEM((ns, H * E), jnp.float32),
        ],
    )(q, k, v, jnp.asarray(bias_s), jnp.asarray(bias_l))
    return out.reshape(1, T, H, E)
